# baseline (speedup 1.0000x reference)
import functools

import jax
import jax.numpy as jnp
from jax import lax
from jax.experimental import pallas as pl
from jax.experimental.pallas import tpu as pltpu
from jax.experimental.pallas import tpu_sc as plsc

_NC = 2
_NS = 16
_NW = _NC * _NS
_CH = 128
_HR = 80
_BM = 400


def _sc_acc(n_rows_pad, width, e_pad):
    per_w = e_pad // _NW
    n_ch = per_w // _CH
    n_pair = n_ch // 2
    rpt = n_rows_pad // _NS
    n_rch = rpt // _CH
    mesh = plsc.VectorSubcoreMesh(core_axis_name="c", subcore_axis_name="s")

    @functools.partial(
        pl.kernel,
        mesh=mesh,
        compiler_params=pltpu.CompilerParams(needs_layout_passes=False),
        out_type=(
            jax.ShapeDtypeStruct((_NC, n_rows_pad, width), jnp.float32),
            jax.ShapeDtypeStruct((_NC, _HR, _CH), jnp.float32),
        ),
        scratch_types=[
            pltpu.VMEM((_CH,), jnp.int32),
            pltpu.VMEM((_CH,), jnp.int32),
            pltpu.VMEM((_CH,), jnp.int32),
            pltpu.VMEM((_CH,), jnp.int32),
            pltpu.VMEM((_HR,), jnp.int32),
            pltpu.VMEM((_HR, _CH), jnp.float32),
            pltpu.VMEM((_CH, width), jnp.float32),
            pltpu.VMEM((_CH, width), jnp.float32),
            pltpu.VMEM_SHARED((n_rows_pad, width), jnp.float32),
            pltpu.VMEM_SHARED((_HR, _CH), jnp.float32),
            pltpu.SemaphoreType.DMA,
            pltpu.SemaphoreType.DMA,
            pltpu.SemaphoreType.DMA,
            pltpu.SemaphoreType.DMA,
            pltpu.SemaphoreType.DMA,
        ],
    )
    def k(feat, srcp, dstp, zw, id_hr, out_acc, out_deg,
          sidx0, sidx1, didx0, didx1, idv, hist, rows0, rows1,
          acc, shist, semg0, semg1, sems0, sems1, semi):
        c = lax.axis_index("c")
        s = lax.axis_index("s")
        wid = s * _NC + c
        r0 = s * rpt
        row0 = wid * n_ch
        pltpu.sync_copy(id_hr, idv)
        pltpu.sync_copy(zw, rows0)
        pltpu.sync_copy(zw.at[pl.ds(0, _HR), :], hist)
        for j in range(n_rch):
            pltpu.sync_copy(rows0, acc.at[pl.ds(r0 + j * _CH, _CH), :])

        @pl.when(s == 0)
        def _():
            pltpu.sync_copy(rows0.at[pl.ds(0, _HR), :], shist)

        plsc.subcore_barrier()

        ones16 = jnp.ones((16,), jnp.float32)

        def hist_update(didx):
            for g in range(_CH // 16):
                iv = didx[pl.ds(g * 16, 16)]
                ivr = lax.shift_right_logical(iv, 7)
                ivc = lax.bitwise_and(iv, 127)
                plsc.addupdate_scatter(hist, [ivr, ivc], ones16)

        pltpu.sync_copy(srcp.at[row0], sidx0)
        pltpu.sync_copy(dstp.at[row0], didx0)
        pltpu.async_copy(feat.at[sidx0], rows0, semg0)

        def body(p, carry):
            i = 2 * p
            pltpu.async_copy(srcp.at[row0 + i + 1], sidx1, semi)
            pltpu.async_copy(dstp.at[row0 + i + 1], didx1, semi)
            pltpu.make_async_copy(feat.at[sidx0], rows0, semg0).wait()
            pltpu.make_async_copy(srcp.at[row0 + i + 1], sidx1,
                                  semi).wait()
            pltpu.make_async_copy(dstp.at[row0 + i + 1], didx1,
                                  semi).wait()
            pltpu.async_copy(feat.at[sidx1], rows1, semg1)
            pltpu.async_copy(rows0, acc.at[didx0], sems0, add=True)
            hist_update(didx0)
            pltpu.make_async_copy(rows0, acc.at[didx0], sems0).wait()

            @pl.when(p < n_pair - 1)
            def _():
                pltpu.async_copy(srcp.at[row0 + i + 2], sidx0, semi)
                pltpu.async_copy(dstp.at[row0 + i + 2], didx0, semi)

            pltpu.make_async_copy(feat.at[sidx1], rows1, semg1).wait()

            @pl.when(p < n_pair - 1)
            def _():
                pltpu.make_async_copy(srcp.at[row0 + i + 2], sidx0,
                                      semi).wait()
                pltpu.make_async_copy(dstp.at[row0 + i + 2], didx0,
                                      semi).wait()
                pltpu.async_copy(feat.at[sidx0], rows0, semg0)

            pltpu.async_copy(rows1, acc.at[didx1], sems1, add=True)
            hist_update(didx1)
            pltpu.make_async_copy(rows1, acc.at[didx1], sems1).wait()
            return carry

        lax.fori_loop(0, n_pair, body, 0)
        plsc.subcore_barrier()
        pltpu.sync_copy(hist, shist.at[idv], add=True)
        plsc.subcore_barrier()
        for j in range(n_rch):
            rj = r0 + j * _CH
            pltpu.sync_copy(acc.at[pl.ds(rj, _CH), :], rows0)
            pltpu.sync_copy(rows0, out_acc.at[c, pl.ds(rj, _CH), :])

        @pl.when(s == 0)
        def _():
            pltpu.sync_copy(shist, rows1.at[pl.ds(0, _HR), :])
            pltpu.sync_copy(rows1.at[pl.ds(0, _HR), :], out_deg.at[c])

    return k


def _mm(x, w, bm):
    n, d = x.shape
    h = w.shape[1]

    def body(x_ref, w_ref, o_ref):
        o_ref[...] = jnp.dot(x_ref[...], w_ref[...],
                             preferred_element_type=jnp.float32)

    return pl.pallas_call(
        body,
        grid=(n // bm,),
        in_specs=[pl.BlockSpec((bm, d), lambda i: (i, 0)),
                  pl.BlockSpec((d, h), lambda i: (0, 0))],
        out_specs=pl.BlockSpec((bm, h), lambda i: (i, 0)),
        out_shape=jax.ShapeDtypeStruct((n, h), jnp.float32),
    )(x, w)


def _layer1_combine(x, acc_a, acc_b, deg_a, deg_b, ws1, b1r, ws2, b2r):
    n, d = x.shape
    c = ws2.shape[1]

    def body(x_ref, aa, ab, da, db, ws1_r, b1_r, ws2_r, b2_r,
             h_ref, hs2_ref):
        agg = aa[...] + ab[...]
        deg = jnp.maximum(da[...] + db[...], 1.0)
        h = jnp.dot(x_ref[...], ws1_r[...],
                    preferred_element_type=jnp.float32)
        h = jnp.maximum(h + b1_r[...] + agg / deg, 0.0)
        h_ref[...] = h
        hs2_ref[...] = jnp.dot(h, ws2_r[...],
                               preferred_element_type=jnp.float32) + b2_r[...]

    return pl.pallas_call(
        body,
        grid=(n // _BM,),
        in_specs=[
            pl.BlockSpec((_BM, d), lambda i: (i, 0)),
            pl.BlockSpec((_BM, d), lambda i: (i, 0)),
            pl.BlockSpec((_BM, d), lambda i: (i, 0)),
            pl.BlockSpec((_BM, 1), lambda i: (i, 0)),
            pl.BlockSpec((_BM, 1), lambda i: (i, 0)),
            pl.BlockSpec((d, d), lambda i: (0, 0)),
            pl.BlockSpec((1, d), lambda i: (0, 0)),
            pl.BlockSpec((d, c), lambda i: (0, 0)),
            pl.BlockSpec((1, c), lambda i: (0, 0)),
        ],
        out_specs=[pl.BlockSpec((_BM, d), lambda i: (i, 0)),
                   pl.BlockSpec((_BM, c), lambda i: (i, 0))],
        out_shape=[jax.ShapeDtypeStruct((n, d), jnp.float32),
                   jax.ShapeDtypeStruct((n, c), jnp.float32)],
    )(x, acc_a, acc_b, deg_a, deg_b, ws1, b1r, ws2, b2r)


def _layer2_combine(hs2, acc_a, acc_b, deg_a, deg_b, wn2):
    n, d = acc_a.shape
    c = wn2.shape[1]

    def body(hs_ref, aa, ab, da, db, wn2_r, o_ref):
        agg = aa[...] + ab[...]
        deg = jnp.maximum(da[...] + db[...], 1.0)
        o_ref[...] = hs_ref[...] + jnp.dot(
            agg / deg, wn2_r[...], preferred_element_type=jnp.float32)

    return pl.pallas_call(
        body,
        grid=(n // _BM,),
        in_specs=[
            pl.BlockSpec((_BM, c), lambda i: (i, 0)),
            pl.BlockSpec((_BM, d), lambda i: (i, 0)),
            pl.BlockSpec((_BM, d), lambda i: (i, 0)),
            pl.BlockSpec((_BM, 1), lambda i: (i, 0)),
            pl.BlockSpec((_BM, 1), lambda i: (i, 0)),
            pl.BlockSpec((d, c), lambda i: (0, 0)),
        ],
        out_specs=pl.BlockSpec((_BM, c), lambda i: (i, 0)),
        out_shape=jax.ShapeDtypeStruct((n, c), jnp.float32),
    )(hs2, acc_a, acc_b, deg_a, deg_b, wn2)


def kernel(x, edge_index1, edge_index2, W_self1, W_neigh1, b1,
           W_self2, W_neigh2, b2):
    n, d = x.shape
    h = W_neigh1.shape[1]
    c = W_neigh2.shape[1]
    e = edge_index1.shape[1]

    quant = _NW * _CH * 2
    e_pad = ((e + quant - 1) // quant) * quant
    rq = _CH * _NS
    rp = ((n + 1 + rq - 1) // rq) * rq
    assert rp <= _HR * _CH

    pad = e_pad - e
    pz = (jnp.arange(pad, dtype=jnp.int32) % n)
    pn = n + (jnp.arange(pad, dtype=jnp.int32) % (rp - n))
    src1 = jnp.concatenate([edge_index1[0], pz]).reshape(-1, _CH)
    dst1 = jnp.concatenate([edge_index1[1], pn]).reshape(-1, _CH)
    src2 = jnp.concatenate([edge_index2[0], pz]).reshape(-1, _CH)
    dst2 = jnp.concatenate([edge_index2[1], pn]).reshape(-1, _CH)

    z_h = jnp.zeros((_CH, h), jnp.float32)
    id_hr = jnp.arange(_HR, dtype=jnp.int32)

    sc_agg = _sc_acc(rp, h, e_pad)

    xw1 = _mm(x, W_neigh1, _BM)
    acc1, deg1 = sc_agg(xw1, src1, dst1, z_h, id_hr)
    deg1 = deg1.reshape(_NC, _HR * _CH)
    h_out, hs2 = _layer1_combine(
        x, acc1[0, :n], acc1[1, :n],
        deg1[0, :n].reshape(n, 1), deg1[1, :n].reshape(n, 1),
        W_self1, b1.reshape(1, h), W_self2, b2.reshape(1, c))

    acc2, deg2 = sc_agg(h_out, src2, dst2, z_h, id_hr)
    deg2 = deg2.reshape(_NC, _HR * _CH)
    out = _layer2_combine(
        hs2, acc2[0, :n], acc2[1, :n],
        deg2[0, :n].reshape(n, 1), deg2[1, :n].reshape(n, 1), W_neigh2)
    return out
